# baseline (speedup 1.0000x reference)
import jax
import jax.numpy as jnp
from jax import lax
from jax.experimental import pallas as pl
from jax.experimental.pallas import tpu as pltpu
from jax.experimental.pallas import tpu_sc as plsc

NC = 2
NS = 16
NW = NC * NS
CHUNK = 128


def _gather_call(num_indices, emb, dtype):
    mesh = plsc.VectorSubcoreMesh(core_axis_name="c", subcore_axis_name="s")
    b_per_w = num_indices // NW
    n_chunks = b_per_w // CHUNK

    @jax.jit
    def run(table, flat_idx):
        @pl.kernel(
            out_type=jax.ShapeDtypeStruct((num_indices, emb), dtype),
            mesh=mesh,
            compiler_params=pltpu.CompilerParams(use_tc_tiling_on_sc=False),
            scratch_types=[
                pltpu.VMEM((CHUNK,), jnp.int32),
                pltpu.VMEM((CHUNK, emb), dtype),
                pltpu.SemaphoreType.DMA,
            ],
        )
        def kern(table_hbm, idx_hbm, out_hbm, idx_v, rows_v, sem):
            wid = lax.axis_index("s") * NC + lax.axis_index("c")
            base = wid * b_per_w

            @pl.loop(0, n_chunks)
            def _(g):
                off = base + g * CHUNK
                pltpu.sync_copy(idx_hbm.at[pl.ds(off, CHUNK)], idx_v)
                pltpu.async_copy(table_hbm.at[idx_v], rows_v, sem).wait()
                pltpu.sync_copy(rows_v, out_hbm.at[pl.ds(off, CHUNK)])

        return kern(table, flat_idx)

    return run


def kernel(input_sentence, table):
    batch, seq = input_sentence.shape
    vocab, emb = table.shape
    num_indices = batch * seq
    flat_idx = input_sentence.reshape(num_indices).astype(jnp.int32)
    run = _gather_call(num_indices, emb, table.dtype)
    out = run(table, flat_idx)
    return out.reshape(batch, seq, emb)


# speedup vs baseline: 1.1936x; 1.1936x over previous
import jax
import jax.numpy as jnp
from jax.experimental import pallas as pl
from jax.experimental.pallas import tpu as pltpu
from jax.experimental.pallas import tpu_sc as plsc

GATHER = 128
WINDOW = 512


def _gather_call(num_indices, emb, dtype):
    mesh = plsc.VectorSubcoreMesh(core_axis_name="c", subcore_axis_name="s")

    @jax.jit
    def run(table, flat_idx):
        @pl.kernel(
            out_type=jax.ShapeDtypeStruct((num_indices, emb), dtype),
            mesh=mesh,
            compiler_params=pltpu.CompilerParams(use_tc_tiling_on_sc=False),
            scratch_types=[pltpu.SemaphoreType.DMA],
        )
        def kern(table_hbm, idx_hbm, out_hbm, sem):
            def body(idx_vmem, out_vmem):
                copies = []
                for j in range(WINDOW // GATHER):
                    copies.append(
                        pltpu.async_copy(
                            table_hbm.at[
                                idx_vmem.at[0, pl.ds(j * GATHER, GATHER)]
                            ],
                            out_vmem.at[pl.ds(j * GATHER, GATHER), :],
                            sem,
                        )
                    )
                for c in copies:
                    c.wait()

            pltpu.emit_pipeline(
                body,
                grid=(num_indices // WINDOW,),
                in_specs=[
                    pl.BlockSpec((1, WINDOW), index_map=lambda i: (0, i))
                ],
                out_specs=[
                    pl.BlockSpec((WINDOW, emb), index_map=lambda i: (i, 0))
                ],
                core_axis_name=("c", "s"),
                dimension_semantics=(pltpu.PARALLEL,),
            )(idx_hbm, out_hbm)

        return kern(table, flat_idx)

    return run


def kernel(input_sentence, table):
    batch, seq = input_sentence.shape
    vocab, emb = table.shape
    num_indices = batch * seq
    flat_idx = input_sentence.reshape(1, num_indices).astype(jnp.int32)
    run = _gather_call(num_indices, emb, table.dtype)
    out = run(table, flat_idx)
    return out.reshape(batch, seq, emb)
